# Initial kernel scaffold; baseline (speedup 1.0000x reference)
import functools

import jax
import jax.numpy as jnp
from jax import lax
from jax.experimental import pallas as pl
from jax.experimental.pallas import tpu as pltpu
from jax.experimental.pallas import tpu_sc as plsc

N = 50000
E = 800000
D = 64
HALF = 32
NSUB = 16
ALPHA = 0.1
LANES = 16

EB = 80
EPW = E // NSUB
NBLK = EPW // EB
RPS = N // NSUB
ZR = 125
CW = 16

_MESH = plsc.VectorSubcoreMesh(core_axis_name="c", subcore_axis_name="s")


def _zero_fill(buf, rows, width):
    @pl.loop(0, rows)
    def _(r):
        for k in range(width // LANES):
            buf[r, pl.ds(k * LANES, LANES)] = jnp.zeros((LANES,), jnp.float32)


def _zero_acc(acc, s, zv):
    @pl.loop(0, RPS // ZR)
    def _(j):
        pltpu.sync_copy(zv, acc.at[pl.ds(s * RPS + j * ZR, ZR)])


@jax.jit
def _degrees(eidx):
    @functools.partial(
        pl.kernel,
        out_type=jax.ShapeDtypeStruct((2 * N, CW), jnp.float32),
        mesh=_MESH,
        scratch_types=[
            pltpu.VMEM_SHARED((N, CW), jnp.float32),
            pltpu.VMEM((EB,), jnp.int32),
            pltpu.VMEM((EB, CW), jnp.float32),
            pltpu.VMEM((ZR, CW), jnp.float32),
        ],
    )
    def k(eidx_hbm, out_hbm, acc, idxv, onesv, zv):
        c = lax.axis_index("c")
        s = lax.axis_index("s")
        _zero_fill(zv, ZR, CW)

        @pl.loop(0, EB)
        def _(r):
            onesv[r, pl.ds(0, LANES)] = jnp.ones((LANES,), jnp.float32)

        _zero_acc(acc, s, zv)
        plsc.subcore_barrier()

        @pl.loop(0, NBLK)
        def _(j):
            e0 = s * EPW + j * EB
            pltpu.sync_copy(eidx_hbm.at[c, pl.ds(e0, EB)], idxv)
            pltpu.sync_copy(onesv, acc.at[idxv], add=True)

        plsc.subcore_barrier()
        pltpu.sync_copy(
            acc.at[pl.ds(s * RPS, RPS)],
            out_hbm.at[pl.ds(c * N + s * RPS, RPS)],
        )

    return k(eidx)


def _spmm_phase(c, s, acc, src_hbm, gidx_hbm, sidx_hbm, out_hbm, gi, si, gb, zv):
    _zero_acc(acc, s, zv)
    plsc.subcore_barrier()

    @pl.loop(0, NBLK)
    def _(j):
        e0 = s * EPW + j * EB
        pltpu.sync_copy(gidx_hbm.at[c, pl.ds(e0, EB)], gi)
        pltpu.sync_copy(sidx_hbm.at[pl.ds(e0, EB)], si)
        pltpu.sync_copy(src_hbm.at[gi], gb)
        pltpu.sync_copy(gb, acc.at[si], add=True)

    plsc.subcore_barrier()
    pltpu.sync_copy(
        acc.at[pl.ds(s * RPS, RPS)],
        out_hbm.at[pl.ds(c * N + s * RPS, RPS)],
    )
    plsc.subcore_barrier()


@jax.jit
def _layer(usrc, isrc, grows, gcols, rows, cols):
    @functools.partial(
        pl.kernel,
        out_type=(
            jax.ShapeDtypeStruct((2 * N, HALF), jnp.float32),
            jax.ShapeDtypeStruct((2 * N, HALF), jnp.float32),
        ),
        mesh=_MESH,
        scratch_types=[
            pltpu.VMEM_SHARED((N, HALF), jnp.float32),
            pltpu.VMEM((EB,), jnp.int32),
            pltpu.VMEM((EB,), jnp.int32),
            pltpu.VMEM((EB, HALF), jnp.float32),
            pltpu.VMEM((ZR, HALF), jnp.float32),
        ],
    )
    def k(usrc_h, isrc_h, grows_h, gcols_h, rows_h, cols_h,
          us_out, is_out, acc, gi, si, gb, zv):
        c = lax.axis_index("c")
        s = lax.axis_index("s")
        _zero_fill(zv, ZR, HALF)
        _spmm_phase(c, s, acc, isrc_h, gcols_h, rows_h, us_out, gi, si, gb, zv)
        _spmm_phase(c, s, acc, usrc_h, grows_h, cols_h, is_out, gi, si, gb, zv)

    return k(usrc, isrc, grows, gcols, rows, cols)


BR = 1000


@jax.jit
def _scale(seg, cnt, base):
    def body(seg_ref, cnt_ref, base_ref, out_ref):
        d = cnt_ref[:, 0:1]
        d = jnp.where(d == 0.0, 1.0, d)
        out_ref[...] = seg_ref[...] / d + ALPHA * base_ref[...]

    return pl.pallas_call(
        body,
        grid=(2, N // BR),
        in_specs=[
            pl.BlockSpec((BR, HALF), lambda c, i: (c * (N // BR) + i, 0)),
            pl.BlockSpec((BR, CW), lambda c, i: (i, 0)),
            pl.BlockSpec((BR, HALF), lambda c, i: (i, c)),
        ],
        out_specs=pl.BlockSpec((BR, HALF), lambda c, i: (c * (N // BR) + i, 0)),
        out_shape=jax.ShapeDtypeStruct((2 * N, HALF), jnp.float32),
    )(seg, cnt, base)


def _unslab(x):
    return jnp.concatenate([x[:N], x[N:]], axis=1)


def kernel(init_user_embeds, init_item_embeds, rows, cols):
    rows32 = rows.astype(jnp.int32)
    cols32 = cols.astype(jnp.int32)
    grows = jnp.stack([rows32, rows32 + N])
    gcols = jnp.stack([cols32, cols32 + N])
    eidx = jnp.stack([rows32, cols32])

    cnt = _degrees(eidx)
    ucnt, icnt = cnt[:N], cnt[N:]

    u0, i0 = init_user_embeds, init_item_embeds
    ucur = jnp.concatenate([u0[:, :HALF], u0[:, HALF:]], axis=0)
    icur = jnp.concatenate([i0[:, :HALF], i0[:, HALF:]], axis=0)

    users, items = [u0], [i0]
    for _ in range(2):
        us, is_ = _layer(ucur, icur, grows, gcols, rows32, cols32)
        ucur = _scale(us, ucnt, u0)
        icur = _scale(is_, icnt, i0)
        users.append(_unslab(ucur))
        items.append(_unslab(icur))
    return jnp.stack(users), jnp.stack(items)


# speedup vs baseline: 4.1079x; 4.1079x over previous
import functools

import jax
import jax.numpy as jnp
from jax import lax
from jax.experimental import pallas as pl
from jax.experimental.pallas import tpu as pltpu
from jax.experimental.pallas import tpu_sc as plsc

N = 50000
NPAD = 50048
E = 800000
D = 64
HALF = 32
NSUB = 16
ALPHA = 0.1
LANES = 16

EB = 80
EPW = E // NSUB
NBLK = EPW // EB
RPS = NPAD // NSUB
ZR = 136
CW = 16

_MESH = plsc.VectorSubcoreMesh(core_axis_name="c", subcore_axis_name="s")
_SC_PARAMS = pltpu.CompilerParams(use_tc_tiling_on_sc=False)


def _zero_fill(buf, rows, width):
    @pl.loop(0, rows)
    def _(r):
        for k in range(width // LANES):
            buf[r, pl.ds(k * LANES, LANES)] = jnp.zeros((LANES,), jnp.float32)


def _zero_acc(acc, s, zv):
    @pl.loop(0, RPS // ZR)
    def _(j):
        pltpu.sync_copy(zv, acc.at[pl.ds(s * RPS + j * ZR, ZR)])


@jax.jit
def _degrees(eidx):
    @functools.partial(
        pl.kernel,
        out_type=jax.ShapeDtypeStruct((2 * NPAD, CW), jnp.float32),
        mesh=_MESH,
        compiler_params=_SC_PARAMS,
        scratch_types=[
            pltpu.VMEM_SHARED((NPAD, CW), jnp.float32),
            pltpu.VMEM((EB,), jnp.int32),
            pltpu.VMEM((EB, CW), jnp.float32),
            pltpu.VMEM((ZR, CW), jnp.float32),
        ],
    )
    def k(eidx_hbm, out_hbm, acc, idxv, onesv, zv):
        c = lax.axis_index("c")
        s = lax.axis_index("s")
        _zero_fill(zv, ZR, CW)

        @pl.loop(0, EB)
        def _(r):
            onesv[r, pl.ds(0, LANES)] = jnp.ones((LANES,), jnp.float32)

        _zero_acc(acc, s, zv)
        plsc.subcore_barrier()

        @pl.loop(0, NBLK)
        def _(j):
            e0 = c * E + s * EPW + j * EB
            pltpu.sync_copy(eidx_hbm.at[pl.ds(e0, EB)], idxv)
            pltpu.sync_copy(onesv, acc.at[idxv], add=True)

        plsc.subcore_barrier()
        pltpu.sync_copy(
            acc.at[pl.ds(s * RPS, RPS)],
            out_hbm.at[pl.ds(c * NPAD + s * RPS, RPS)],
        )

    return k(eidx)


def _spmm_phase(c, s, acc, src_hbm, gidx_hbm, sidx_hbm, out_hbm, gi, si, gb, zv):
    _zero_acc(acc, s, zv)
    plsc.subcore_barrier()

    @pl.loop(0, NBLK)
    def _(j):
        e0 = s * EPW + j * EB
        pltpu.sync_copy(gidx_hbm.at[pl.ds(c * E + e0, EB)], gi)
        pltpu.sync_copy(sidx_hbm.at[pl.ds(e0, EB)], si)
        pltpu.sync_copy(src_hbm.at[gi], gb)
        pltpu.sync_copy(gb, acc.at[si], add=True)

    plsc.subcore_barrier()
    pltpu.sync_copy(
        acc.at[pl.ds(s * RPS, RPS)],
        out_hbm.at[pl.ds(c * NPAD + s * RPS, RPS)],
    )
    plsc.subcore_barrier()


@jax.jit
def _layer(usrc, isrc, grows, gcols, rows, cols):
    @functools.partial(
        pl.kernel,
        out_type=(
            jax.ShapeDtypeStruct((2 * NPAD, HALF), jnp.float32),
            jax.ShapeDtypeStruct((2 * NPAD, HALF), jnp.float32),
        ),
        mesh=_MESH,
        compiler_params=_SC_PARAMS,
        scratch_types=[
            pltpu.VMEM_SHARED((NPAD, HALF), jnp.float32),
            pltpu.VMEM((EB,), jnp.int32),
            pltpu.VMEM((EB,), jnp.int32),
            pltpu.VMEM((EB, HALF), jnp.float32),
            pltpu.VMEM((ZR, HALF), jnp.float32),
        ],
    )
    def k(usrc_h, isrc_h, grows_h, gcols_h, rows_h, cols_h,
          us_out, is_out, acc, gi, si, gb, zv):
        c = lax.axis_index("c")
        s = lax.axis_index("s")
        _zero_fill(zv, ZR, HALF)
        _spmm_phase(c, s, acc, isrc_h, gcols_h, rows_h, us_out, gi, si, gb, zv)
        _spmm_phase(c, s, acc, usrc_h, grows_h, cols_h, is_out, gi, si, gb, zv)

    return k(usrc, isrc, grows, gcols, rows, cols)


BR = 6256


@jax.jit
def _scale(seg, cnt, base):
    def body(seg_ref, cnt_ref, base_ref, out_ref):
        d = cnt_ref[:, 0:1]
        d = jnp.where(d == 0.0, 1.0, d)
        out_ref[...] = seg_ref[...] / d + ALPHA * base_ref[...]

    return pl.pallas_call(
        body,
        grid=(2, NPAD // BR),
        in_specs=[
            pl.BlockSpec((BR, HALF), lambda c, i: (c * (NPAD // BR) + i, 0)),
            pl.BlockSpec((BR, CW), lambda c, i: (i, 0)),
            pl.BlockSpec((BR, HALF), lambda c, i: (c * (NPAD // BR) + i, 0)),
        ],
        out_specs=pl.BlockSpec((BR, HALF), lambda c, i: (c * (NPAD // BR) + i, 0)),
        out_shape=jax.ShapeDtypeStruct((2 * NPAD, HALF), jnp.float32),
    )(seg, cnt, base)


def _unslab(x):
    return jnp.concatenate([x[:N], x[NPAD:NPAD + N]], axis=1)


def kernel(init_user_embeds, init_item_embeds, rows, cols):
    rows32 = rows.astype(jnp.int32)
    cols32 = cols.astype(jnp.int32)
    grows = jnp.concatenate([rows32, rows32 + NPAD])
    gcols = jnp.concatenate([cols32, cols32 + NPAD])
    eidx = jnp.concatenate([rows32, cols32])

    cnt = _degrees(eidx)
    ucnt, icnt = cnt[:NPAD], cnt[NPAD:]

    u0, i0 = init_user_embeds, init_item_embeds
    pad = ((0, NPAD - N), (0, 0))
    u0p = jnp.pad(u0, pad)
    i0p = jnp.pad(i0, pad)
    u0s = jnp.concatenate([u0p[:, :HALF], u0p[:, HALF:]], axis=0)
    i0s = jnp.concatenate([i0p[:, :HALF], i0p[:, HALF:]], axis=0)
    ucur, icur = u0s, i0s

    users, items = [u0], [i0]
    for _ in range(2):
        us, is_ = _layer(ucur, icur, grows, gcols, rows32, cols32)
        ucur = _scale(us, ucnt, u0s)
        icur = _scale(is_, icnt, i0s)
        users.append(_unslab(ucur))
        items.append(_unslab(icur))
    return jnp.stack(users), jnp.stack(items)


# speedup vs baseline: 8.8069x; 2.1439x over previous
import functools

import jax
import jax.numpy as jnp
from jax import lax
from jax.experimental import pallas as pl
from jax.experimental.pallas import tpu as pltpu
from jax.experimental.pallas import tpu_sc as plsc

N = 50000
NPAD = 50048
E = 800000
D = 64
HALF = 32
NSUB = 16
ALPHA = 0.1
LANES = 16

EB = 80
NBLK = 628
EPW = NBLK * EB
EPAD = NSUB * EPW
RPS = NPAD // NSUB
ZR = 136
CW = 16

_MESH = plsc.VectorSubcoreMesh(core_axis_name="c", subcore_axis_name="s")
_SC_PARAMS = pltpu.CompilerParams(use_tc_tiling_on_sc=False)


def _zero_fill(buf, rows, width):
    @pl.loop(0, rows)
    def _(r):
        for k in range(width // LANES):
            buf[r, pl.ds(k * LANES, LANES)] = jnp.zeros((LANES,), jnp.float32)


def _zero_acc(acc, s, zv):
    @pl.loop(0, RPS // ZR)
    def _(j):
        pltpu.sync_copy(zv, acc.at[pl.ds(s * RPS + j * ZR, ZR)])


@jax.jit
def _degrees(eidx):
    @functools.partial(
        pl.kernel,
        out_type=jax.ShapeDtypeStruct((2 * NPAD, CW), jnp.float32),
        mesh=_MESH,
        compiler_params=_SC_PARAMS,
        scratch_types=[
            pltpu.VMEM_SHARED((NPAD, CW), jnp.float32),
            pltpu.VMEM((EB,), jnp.int32),
            pltpu.VMEM((EB, CW), jnp.float32),
            pltpu.VMEM((ZR, CW), jnp.float32),
        ],
    )
    def k(eidx_hbm, out_hbm, acc, idxv, onesv, zv):
        c = lax.axis_index("c")
        s = lax.axis_index("s")
        _zero_fill(zv, ZR, CW)

        @pl.loop(0, EB)
        def _(r):
            onesv[r, pl.ds(0, LANES)] = jnp.ones((LANES,), jnp.float32)

        _zero_acc(acc, s, zv)
        plsc.subcore_barrier()

        @pl.loop(0, NBLK)
        def _(j):
            e0 = c * EPAD + s * EPW + j * EB
            pltpu.sync_copy(eidx_hbm.at[pl.ds(e0, EB)], idxv)
            pltpu.sync_copy(onesv, acc.at[idxv], add=True)

        plsc.subcore_barrier()
        pltpu.sync_copy(
            acc.at[pl.ds(s * RPS, RPS)],
            out_hbm.at[pl.ds(c * NPAD + s * RPS, RPS)],
        )

    return k(eidx)


def _spmm_phase(c, s, acc, src_hbm, gidx_hbm, sidx_hbm, out_hbm,
                gi, si, gb, zv, gisem, sisem, gsem):
    _zero_acc(acc, s, zv)
    plsc.subcore_barrier()
    ebase = s * EPW
    gbase = c * EPAD + s * EPW

    def issue_idx(j, q):
        e0 = j * EB
        pltpu.async_copy(gidx_hbm.at[pl.ds(gbase + e0, EB)], gi[q], gisem.at[q])
        pltpu.async_copy(sidx_hbm.at[pl.ds(ebase + e0, EB)], si[q], sisem.at[q])

    def wait_idx(j, q):
        pltpu.make_async_copy(gidx_hbm.at[pl.ds(gbase, EB)], gi[q], gisem.at[q]).wait()
        pltpu.make_async_copy(sidx_hbm.at[pl.ds(ebase, EB)], si[q], sisem.at[q]).wait()

    def issue_gather(q, p):
        pltpu.async_copy(src_hbm.at[gi[q]], gb[p], gsem.at[p])

    def wait_gather(q, p):
        pltpu.make_async_copy(src_hbm.at[gi[q]], gb[p], gsem.at[p]).wait()

    for q in range(4):
        issue_idx(q, q)
    wait_idx(0, 0)
    issue_gather(0, 0)

    def step(j, jj):
        qn = (jj + 1) % 4
        wait_idx(j + 1, qn)
        issue_gather(qn, (jj + 1) % 2)
        wait_gather(jj % 4, jj % 2)
        pltpu.sync_copy(gb[jj % 2], acc.at[si[jj % 4]], add=True)
        issue_idx(j + 4, jj % 4)

    @pl.loop(0, NBLK - 4, step=4)
    def _(j):
        for u in range(4):
            step(j + u, u)

    for u in range(4):
        j = NBLK - 4 + u
        jj = j % 4
        if u < 3:
            qn = (jj + 1) % 4
            wait_idx(j + 1, qn)
            issue_gather(qn, (jj + 1) % 2)
        wait_gather(jj % 4, jj % 2)
        pltpu.sync_copy(gb[jj % 2], acc.at[si[jj % 4]], add=True)

    plsc.subcore_barrier()
    pltpu.sync_copy(
        acc.at[pl.ds(s * RPS, RPS)],
        out_hbm.at[pl.ds(c * NPAD + s * RPS, RPS)],
    )
    plsc.subcore_barrier()


@jax.jit
def _layer(usrc, isrc, grows, gcols, rows, cols):
    @functools.partial(
        pl.kernel,
        out_type=(
            jax.ShapeDtypeStruct((2 * NPAD, HALF), jnp.float32),
            jax.ShapeDtypeStruct((2 * NPAD, HALF), jnp.float32),
        ),
        mesh=_MESH,
        compiler_params=_SC_PARAMS,
        scratch_types=[
            pltpu.VMEM_SHARED((NPAD, HALF), jnp.float32),
            [pltpu.VMEM((EB,), jnp.int32)] * 4,
            [pltpu.VMEM((EB,), jnp.int32)] * 4,
            [pltpu.VMEM((EB, HALF), jnp.float32)] * 2,
            pltpu.VMEM((ZR, HALF), jnp.float32),
            pltpu.SemaphoreType.DMA((4,)),
            pltpu.SemaphoreType.DMA((4,)),
            pltpu.SemaphoreType.DMA((2,)),
        ],
    )
    def k(usrc_h, isrc_h, grows_h, gcols_h, rows_h, cols_h,
          us_out, is_out, acc, gi, si, gb, zv, gisem, sisem, gsem):
        c = lax.axis_index("c")
        s = lax.axis_index("s")
        _zero_fill(zv, ZR, HALF)
        _spmm_phase(c, s, acc, isrc_h, gcols_h, rows_h, us_out,
                    gi, si, gb, zv, gisem, sisem, gsem)
        _spmm_phase(c, s, acc, usrc_h, grows_h, cols_h, is_out,
                    gi, si, gb, zv, gisem, sisem, gsem)

    return k(usrc, isrc, grows, gcols, rows, cols)


BR = 6256


@jax.jit
def _scale(seg, cnt, base):
    def body(seg_ref, cnt_ref, base_ref, out_ref):
        d = cnt_ref[:, 0:1]
        d = jnp.where(d == 0.0, 1.0, d)
        out_ref[...] = seg_ref[...] / d + ALPHA * base_ref[...]

    return pl.pallas_call(
        body,
        grid=(2, NPAD // BR),
        in_specs=[
            pl.BlockSpec((BR, HALF), lambda c, i: (c * (NPAD // BR) + i, 0)),
            pl.BlockSpec((BR, CW), lambda c, i: (i, 0)),
            pl.BlockSpec((BR, HALF), lambda c, i: (c * (NPAD // BR) + i, 0)),
        ],
        out_specs=pl.BlockSpec((BR, HALF), lambda c, i: (c * (NPAD // BR) + i, 0)),
        out_shape=jax.ShapeDtypeStruct((2 * NPAD, HALF), jnp.float32),
    )(seg, cnt, base)


def _unslab(x):
    return jnp.concatenate([x[:N], x[NPAD:NPAD + N]], axis=1)


def kernel(init_user_embeds, init_item_embeds, rows, cols):
    rows32 = rows.astype(jnp.int32)
    cols32 = cols.astype(jnp.int32)
    epad = jnp.full((EPAD - E,), N, jnp.int32)
    rows_p = jnp.concatenate([rows32, epad])
    cols_p = jnp.concatenate([cols32, epad])
    grows = jnp.concatenate([rows_p, rows_p + NPAD])
    gcols = jnp.concatenate([cols_p, cols_p + NPAD])
    eidx = jnp.concatenate([rows_p, cols_p])

    cnt = _degrees(eidx)
    ucnt, icnt = cnt[:NPAD], cnt[NPAD:]

    u0, i0 = init_user_embeds, init_item_embeds
    pad = ((0, NPAD - N), (0, 0))
    u0p = jnp.pad(u0, pad)
    i0p = jnp.pad(i0, pad)
    u0s = jnp.concatenate([u0p[:, :HALF], u0p[:, HALF:]], axis=0)
    i0s = jnp.concatenate([i0p[:, :HALF], i0p[:, HALF:]], axis=0)
    ucur, icur = u0s, i0s

    users, items = [u0], [i0]
    for _ in range(2):
        us, is_ = _layer(ucur, icur, grows, gcols, rows_p, cols_p)
        ucur = _scale(us, ucnt, u0s)
        icur = _scale(is_, icnt, i0s)
        users.append(_unslab(ucur))
        items.append(_unslab(icur))
    return jnp.stack(users), jnp.stack(items)


# speedup vs baseline: 12.0906x; 1.3729x over previous
import functools

import jax
import jax.numpy as jnp
from jax import lax
from jax.experimental import pallas as pl
from jax.experimental.pallas import tpu as pltpu
from jax.experimental.pallas import tpu_sc as plsc

N = 50000
NPAD = 50048
E = 800000
D = 64
HALF = 32
NSUB = 16
ALPHA = 0.1
LANES = 16

EB = 128
NBLK = 392
EPW = NBLK * EB
EPAD = NSUB * EPW
IDEPTH = 8
GDEPTH = 4
LOOK = 3
RPS = NPAD // NSUB
ZR = 136
CW = 16

_MESH = plsc.VectorSubcoreMesh(core_axis_name="c", subcore_axis_name="s")
_SC_PARAMS = pltpu.CompilerParams(use_tc_tiling_on_sc=False)


def _zero_fill(buf, rows, width):
    @pl.loop(0, rows)
    def _(r):
        for k in range(width // LANES):
            buf[r, pl.ds(k * LANES, LANES)] = jnp.zeros((LANES,), jnp.float32)


def _zero_acc(acc, s, zv):
    @pl.loop(0, RPS // ZR)
    def _(j):
        pltpu.sync_copy(zv, acc.at[pl.ds(s * RPS + j * ZR, ZR)])


@jax.jit
def _degrees(eidx):
    @functools.partial(
        pl.kernel,
        out_type=jax.ShapeDtypeStruct((2 * NPAD, CW), jnp.float32),
        mesh=_MESH,
        compiler_params=_SC_PARAMS,
        scratch_types=[
            pltpu.VMEM_SHARED((NPAD, CW), jnp.float32),
            pltpu.VMEM((EB,), jnp.int32),
            pltpu.VMEM((EB, CW), jnp.float32),
            pltpu.VMEM((ZR, CW), jnp.float32),
        ],
    )
    def k(eidx_hbm, out_hbm, acc, idxv, onesv, zv):
        c = lax.axis_index("c")
        s = lax.axis_index("s")
        _zero_fill(zv, ZR, CW)

        @pl.loop(0, EB)
        def _(r):
            onesv[r, pl.ds(0, LANES)] = jnp.ones((LANES,), jnp.float32)

        _zero_acc(acc, s, zv)
        plsc.subcore_barrier()

        @pl.loop(0, NBLK)
        def _(j):
            e0 = c * EPAD + s * EPW + j * EB
            pltpu.sync_copy(eidx_hbm.at[pl.ds(e0, EB)], idxv)
            pltpu.sync_copy(onesv, acc.at[idxv], add=True)

        plsc.subcore_barrier()
        pltpu.sync_copy(
            acc.at[pl.ds(s * RPS, RPS)],
            out_hbm.at[pl.ds(c * NPAD + s * RPS, RPS)],
        )

    return k(eidx)


def _spmm_phase(c, s, acc, src_hbm, gidx_hbm, sidx_hbm, out_hbm,
                gi, si, gb, zv, gisem, sisem, gsem):
    _zero_acc(acc, s, zv)
    plsc.subcore_barrier()
    ebase = s * EPW
    gbase = c * EPAD + s * EPW

    def issue_idx(j, q):
        e0 = j * EB
        pltpu.async_copy(gidx_hbm.at[pl.ds(gbase + e0, EB)], gi[q], gisem.at[q])
        pltpu.async_copy(sidx_hbm.at[pl.ds(ebase + e0, EB)], si[q], sisem.at[q])

    def wait_idx(q):
        pltpu.make_async_copy(gidx_hbm.at[pl.ds(gbase, EB)], gi[q], gisem.at[q]).wait()
        pltpu.make_async_copy(sidx_hbm.at[pl.ds(ebase, EB)], si[q], sisem.at[q]).wait()

    def issue_gather(q, p):
        pltpu.async_copy(src_hbm.at[gi[q]], gb[p], gsem.at[p])

    def wait_gather(q, p):
        pltpu.make_async_copy(src_hbm.at[gi[q]], gb[p], gsem.at[p]).wait()

    for q in range(IDEPTH):
        issue_idx(q, q)
    for j in range(LOOK):
        wait_idx(j)
        issue_gather(j, j % GDEPTH)

    def step(j, u, prefetch, advance):
        if advance:
            wait_idx((u + LOOK) % IDEPTH)
            issue_gather((u + LOOK) % IDEPTH, (u + LOOK) % GDEPTH)
        wait_gather(u % IDEPTH, u % GDEPTH)
        pltpu.sync_copy(gb[u % GDEPTH], acc.at[si[u % IDEPTH]], add=True)
        if prefetch:
            issue_idx(j + IDEPTH, u % IDEPTH)

    @pl.loop(0, NBLK - IDEPTH, step=IDEPTH)
    def _(j):
        for u in range(IDEPTH):
            step(j + u, u, True, True)

    for u in range(IDEPTH):
        j = NBLK - IDEPTH + u
        step(j, j % IDEPTH, False, j + LOOK < NBLK)

    plsc.subcore_barrier()
    pltpu.sync_copy(
        acc.at[pl.ds(s * RPS, RPS)],
        out_hbm.at[pl.ds(c * NPAD + s * RPS, RPS)],
    )
    plsc.subcore_barrier()


@jax.jit
def _layer(usrc, isrc, grows, gcols, rows, cols):
    @functools.partial(
        pl.kernel,
        out_type=(
            jax.ShapeDtypeStruct((2 * NPAD, HALF), jnp.float32),
            jax.ShapeDtypeStruct((2 * NPAD, HALF), jnp.float32),
        ),
        mesh=_MESH,
        compiler_params=_SC_PARAMS,
        scratch_types=[
            pltpu.VMEM_SHARED((NPAD, HALF), jnp.float32),
            [pltpu.VMEM((EB,), jnp.int32)] * IDEPTH,
            [pltpu.VMEM((EB,), jnp.int32)] * IDEPTH,
            [pltpu.VMEM((EB, HALF), jnp.float32)] * GDEPTH,
            pltpu.VMEM((ZR, HALF), jnp.float32),
            pltpu.SemaphoreType.DMA((IDEPTH,)),
            pltpu.SemaphoreType.DMA((IDEPTH,)),
            pltpu.SemaphoreType.DMA((GDEPTH,)),
        ],
    )
    def k(usrc_h, isrc_h, grows_h, gcols_h, rows_h, cols_h,
          us_out, is_out, acc, gi, si, gb, zv, gisem, sisem, gsem):
        c = lax.axis_index("c")
        s = lax.axis_index("s")
        _zero_fill(zv, ZR, HALF)
        _spmm_phase(c, s, acc, isrc_h, gcols_h, rows_h, us_out,
                    gi, si, gb, zv, gisem, sisem, gsem)
        _spmm_phase(c, s, acc, usrc_h, grows_h, cols_h, is_out,
                    gi, si, gb, zv, gisem, sisem, gsem)

    return k(usrc, isrc, grows, gcols, rows, cols)


BR = 6256


@jax.jit
def _scale(seg, cnt, base):
    def body(seg_ref, cnt_ref, base_ref, out_ref):
        d = cnt_ref[:, 0:1]
        d = jnp.where(d == 0.0, 1.0, d)
        out_ref[...] = seg_ref[...] / d + ALPHA * base_ref[...]

    return pl.pallas_call(
        body,
        grid=(2, NPAD // BR),
        in_specs=[
            pl.BlockSpec((BR, HALF), lambda c, i: (c * (NPAD // BR) + i, 0)),
            pl.BlockSpec((BR, CW), lambda c, i: (i, 0)),
            pl.BlockSpec((BR, HALF), lambda c, i: (c * (NPAD // BR) + i, 0)),
        ],
        out_specs=pl.BlockSpec((BR, HALF), lambda c, i: (c * (NPAD // BR) + i, 0)),
        out_shape=jax.ShapeDtypeStruct((2 * NPAD, HALF), jnp.float32),
    )(seg, cnt, base)


def _unslab(x):
    return jnp.concatenate([x[:N], x[NPAD:NPAD + N]], axis=1)


def kernel(init_user_embeds, init_item_embeds, rows, cols):
    rows32 = rows.astype(jnp.int32)
    cols32 = cols.astype(jnp.int32)
    epad = jnp.full((EPAD - E,), N, jnp.int32)
    rows_p = jnp.concatenate([rows32, epad])
    cols_p = jnp.concatenate([cols32, epad])
    grows = jnp.concatenate([rows_p, rows_p + NPAD])
    gcols = jnp.concatenate([cols_p, cols_p + NPAD])
    eidx = jnp.concatenate([rows_p, cols_p])

    cnt = _degrees(eidx)
    ucnt, icnt = cnt[:NPAD], cnt[NPAD:]

    u0, i0 = init_user_embeds, init_item_embeds
    pad = ((0, NPAD - N), (0, 0))
    u0p = jnp.pad(u0, pad)
    i0p = jnp.pad(i0, pad)
    u0s = jnp.concatenate([u0p[:, :HALF], u0p[:, HALF:]], axis=0)
    i0s = jnp.concatenate([i0p[:, :HALF], i0p[:, HALF:]], axis=0)
    ucur, icur = u0s, i0s

    users, items = [u0], [i0]
    for _ in range(2):
        us, is_ = _layer(ucur, icur, grows, gcols, rows_p, cols_p)
        ucur = _scale(us, ucnt, u0s)
        icur = _scale(is_, icnt, i0s)
        users.append(_unslab(ucur))
        items.append(_unslab(icur))
    return jnp.stack(users), jnp.stack(items)


# speedup vs baseline: 12.8661x; 1.0641x over previous
import functools

import jax
import jax.numpy as jnp
from jax import lax
from jax.experimental import pallas as pl
from jax.experimental.pallas import tpu as pltpu
from jax.experimental.pallas import tpu_sc as plsc

N = 50000
NPAD = 50048
E = 800000
D = 64
HALF = 32
NSUB = 16
ALPHA = 0.1
LANES = 16

EB = 128
NBLK = 392
EPW = NBLK * EB
EPAD = NSUB * EPW
IDEPTH = 8
GDEPTH = 4
LOOK = 3
RPS = NPAD // NSUB
ZR = 136
CW = 16

_MESH = plsc.VectorSubcoreMesh(core_axis_name="c", subcore_axis_name="s")
_SC_PARAMS = pltpu.CompilerParams(use_tc_tiling_on_sc=False)


def _zero_fill(buf, rows, width):
    @pl.loop(0, rows)
    def _(r):
        for k in range(width // LANES):
            buf[r, pl.ds(k * LANES, LANES)] = jnp.zeros((LANES,), jnp.float32)


def _zero_acc(acc, s, zv):
    @pl.loop(0, RPS // ZR)
    def _(j):
        pltpu.sync_copy(zv, acc.at[pl.ds(s * RPS + j * ZR, ZR)])


def _deg_phase(c, s, acc, eidx_hbm, out_hbm, si, onesv, zv, sisem, scsem):
    _zero_acc(acc, s, zv)

    @pl.loop(0, EB)
    def _(r):
        for k in range(HALF // LANES):
            onesv[r, pl.ds(k * LANES, LANES)] = jnp.ones((LANES,), jnp.float32)

    plsc.subcore_barrier()
    ebase = c * EPAD + s * EPW

    def issue_idx(j, q):
        pltpu.async_copy(eidx_hbm.at[pl.ds(ebase + j * EB, EB)], si[q], sisem.at[q])

    def wait_idx(q):
        pltpu.make_async_copy(eidx_hbm.at[pl.ds(ebase, EB)], si[q], sisem.at[q]).wait()

    def step(j, u, do_prefetch):
        wait_idx(u % IDEPTH)
        pltpu.sync_copy(onesv, acc.at[si[u % IDEPTH]], add=True)
        if do_prefetch:
            issue_idx(j + IDEPTH, u % IDEPTH)

    for q in range(IDEPTH):
        issue_idx(q, q)

    @pl.loop(0, NBLK - IDEPTH, step=IDEPTH)
    def _(j):
        for u in range(IDEPTH):
            step(j + u, u, True)

    for u in range(IDEPTH):
        step(NBLK - IDEPTH + u, u, False)

    plsc.subcore_barrier()
    pltpu.sync_copy(
        acc.at[pl.ds(s * RPS, RPS)],
        out_hbm.at[pl.ds(c * NPAD + s * RPS, RPS)],
    )
    plsc.subcore_barrier()


def _spmm_phase(c, s, acc, src_hbm, gidx_hbm, sidx_hbm, out_hbm,
                gi, si, gb, zv, gisem, sisem, gsem):
    _zero_acc(acc, s, zv)
    plsc.subcore_barrier()
    ebase = s * EPW
    gbase = c * EPAD + s * EPW

    def issue_idx(j, q):
        e0 = j * EB
        pltpu.async_copy(gidx_hbm.at[pl.ds(gbase + e0, EB)], gi[q], gisem.at[q])
        pltpu.async_copy(sidx_hbm.at[pl.ds(ebase + e0, EB)], si[q], sisem.at[q])

    def wait_idx(q):
        pltpu.make_async_copy(gidx_hbm.at[pl.ds(gbase, EB)], gi[q], gisem.at[q]).wait()
        pltpu.make_async_copy(sidx_hbm.at[pl.ds(ebase, EB)], si[q], sisem.at[q]).wait()

    def issue_gather(q, p):
        pltpu.async_copy(src_hbm.at[gi[q]], gb[p], gsem.at[p])

    def wait_gather(q, p):
        pltpu.make_async_copy(src_hbm.at[gi[q]], gb[p], gsem.at[p]).wait()

    for q in range(IDEPTH):
        issue_idx(q, q)
    for j in range(LOOK):
        wait_idx(j)
        issue_gather(j, j % GDEPTH)

    def step(j, u, prefetch, advance):
        if advance:
            wait_idx((u + LOOK) % IDEPTH)
            issue_gather((u + LOOK) % IDEPTH, (u + LOOK) % GDEPTH)
        wait_gather(u % IDEPTH, u % GDEPTH)
        pltpu.sync_copy(gb[u % GDEPTH], acc.at[si[u % IDEPTH]], add=True)
        if prefetch:
            issue_idx(j + IDEPTH, u % IDEPTH)

    @pl.loop(0, NBLK - IDEPTH, step=IDEPTH)
    def _(j):
        for u in range(IDEPTH):
            step(j + u, u, True, True)

    for u in range(IDEPTH):
        j = NBLK - IDEPTH + u
        step(j, j % IDEPTH, False, j + LOOK < NBLK)

    plsc.subcore_barrier()
    pltpu.sync_copy(
        acc.at[pl.ds(s * RPS, RPS)],
        out_hbm.at[pl.ds(c * NPAD + s * RPS, RPS)],
    )
    plsc.subcore_barrier()


@functools.partial(jax.jit, static_argnums=(0,))
def _layer(with_deg, usrc, isrc, grows, gcols, rows, cols, eidx):
    outs = [
        jax.ShapeDtypeStruct((2 * NPAD, HALF), jnp.float32),
        jax.ShapeDtypeStruct((2 * NPAD, HALF), jnp.float32),
    ]
    if with_deg:
        outs.append(jax.ShapeDtypeStruct((2 * NPAD, HALF), jnp.float32))

    @functools.partial(
        pl.kernel,
        out_type=tuple(outs),
        mesh=_MESH,
        compiler_params=_SC_PARAMS,
        scratch_types=[
            pltpu.VMEM_SHARED((NPAD, HALF), jnp.float32),
            [pltpu.VMEM((EB,), jnp.int32)] * IDEPTH,
            [pltpu.VMEM((EB,), jnp.int32)] * IDEPTH,
            [pltpu.VMEM((EB, HALF), jnp.float32)] * GDEPTH,
            pltpu.VMEM((EB, HALF), jnp.float32),
            pltpu.VMEM((ZR, HALF), jnp.float32),
            pltpu.SemaphoreType.DMA((IDEPTH,)),
            pltpu.SemaphoreType.DMA((IDEPTH,)),
            pltpu.SemaphoreType.DMA((GDEPTH,)),
            pltpu.SemaphoreType.DMA((GDEPTH,)),
        ],
    )
    def k(usrc_h, isrc_h, grows_h, gcols_h, rows_h, cols_h, eidx_h,
          *refs):
        out_refs, (acc, gi, si, gb, onesv, zv, gisem, sisem, gsem, scsem) = (
            refs[:-10], refs[-10:])
        c = lax.axis_index("c")
        s = lax.axis_index("s")
        _zero_fill(zv, ZR, HALF)
        if with_deg:
            us_out, is_out, cnt_out = out_refs
            _deg_phase(c, s, acc, eidx_h, cnt_out, si, onesv, zv, sisem, scsem)
        else:
            us_out, is_out = out_refs
        _spmm_phase(c, s, acc, isrc_h, gcols_h, rows_h, us_out,
                    gi, si, gb, zv, gisem, sisem, gsem)
        _spmm_phase(c, s, acc, usrc_h, grows_h, cols_h, is_out,
                    gi, si, gb, zv, gisem, sisem, gsem)

    return k(usrc, isrc, grows, gcols, rows, cols, eidx)


BR = 6256


@jax.jit
def _scale(seg, cnt, base):
    def body(seg_ref, cnt_ref, base_ref, out_ref):
        d = cnt_ref[:, 0:1]
        d = jnp.where(d == 0.0, 1.0, d)
        out_ref[...] = seg_ref[...] / d + ALPHA * base_ref[...]

    return pl.pallas_call(
        body,
        grid=(2, NPAD // BR),
        in_specs=[
            pl.BlockSpec((BR, HALF), lambda c, i: (c * (NPAD // BR) + i, 0)),
            pl.BlockSpec((BR, HALF), lambda c, i: (i, 0)),
            pl.BlockSpec((BR, HALF), lambda c, i: (c * (NPAD // BR) + i, 0)),
        ],
        out_specs=pl.BlockSpec((BR, HALF), lambda c, i: (c * (NPAD // BR) + i, 0)),
        out_shape=jax.ShapeDtypeStruct((2 * NPAD, HALF), jnp.float32),
    )(seg, cnt, base)


def _unslab(x):
    return jnp.concatenate([x[:N], x[NPAD:NPAD + N]], axis=1)


def kernel(init_user_embeds, init_item_embeds, rows, cols):
    rows32 = rows.astype(jnp.int32)
    cols32 = cols.astype(jnp.int32)
    epad = jnp.full((EPAD - E,), N, jnp.int32)
    rows_p = jnp.concatenate([rows32, epad])
    cols_p = jnp.concatenate([cols32, epad])
    grows = jnp.concatenate([rows_p, rows_p + NPAD])
    gcols = jnp.concatenate([cols_p, cols_p + NPAD])
    eidx = jnp.concatenate([rows_p, cols_p])

    u0, i0 = init_user_embeds, init_item_embeds
    pad = ((0, NPAD - N), (0, 0))
    u0p = jnp.pad(u0, pad)
    i0p = jnp.pad(i0, pad)
    u0s = jnp.concatenate([u0p[:, :HALF], u0p[:, HALF:]], axis=0)
    i0s = jnp.concatenate([i0p[:, :HALF], i0p[:, HALF:]], axis=0)
    ucur, icur = u0s, i0s

    users, items = [u0], [i0]
    us, is_, cnt = _layer(True, ucur, icur, grows, gcols, rows_p, cols_p, eidx)
    ucnt, icnt = cnt[:NPAD], cnt[NPAD:]
    for layer in range(2):
        if layer > 0:
            us, is_ = _layer(False, ucur, icur, grows, gcols, rows_p, cols_p,
                             eidx)
        ucur = _scale(us, ucnt, u0s)
        icur = _scale(is_, icnt, i0s)
        users.append(_unslab(ucur))
        items.append(_unslab(icur))
    return jnp.stack(users), jnp.stack(items)


# speedup vs baseline: 19.1920x; 1.4917x over previous
import functools

import jax
import jax.numpy as jnp
from jax import lax
from jax.experimental import pallas as pl
from jax.experimental.pallas import tpu as pltpu
from jax.experimental.pallas import tpu_sc as plsc

N = 50000
NPAD = 50048
E = 800000
D = 64
HALF = 32
NSUB = 16
ALPHA = 0.1
LANES = 16

EB = 128
NBLK = 392
EPW = NBLK * EB
EPAD = NSUB * EPW
IDEPTH = 8
GDEPTH = 4
LOOK = 3
RPS = NPAD // NSUB
ZR = 136

_MESH = plsc.VectorSubcoreMesh(core_axis_name="c", subcore_axis_name="s")
_SC_PARAMS = pltpu.CompilerParams(use_tc_tiling_on_sc=False)


def _zero_fill(buf, rows, width):
    @pl.loop(0, rows)
    def _(r):
        for k in range(width // LANES):
            buf[r, pl.ds(k * LANES, LANES)] = jnp.zeros((LANES,), jnp.float32)


def _zero_acc(acc, s, zv):
    @pl.loop(0, RPS // ZR)
    def _(j):
        pltpu.sync_copy(zv, acc.at[pl.ds(s * RPS + j * ZR, ZR)])


def _xform_idx(gi_q, c):
    for k in range(EB // LANES):
        v = gi_q[pl.ds(k * LANES, LANES)]
        gi_q[pl.ds(k * LANES, LANES)] = v + v + c


def _dump_acc(acc, out_hbm, c, s):
    pltpu.sync_copy(
        acc.at[pl.ds(s * RPS, RPS)],
        out_hbm.at[pl.ds(s * RPS, RPS), c],
    )


def _deg_phase(c, s, acc, eidx_hbm, out_hbm, si, onesv, zv, sisem):
    _zero_acc(acc, s, zv)

    @pl.loop(0, EB)
    def _(r):
        for k in range(HALF // LANES):
            onesv[r, pl.ds(k * LANES, LANES)] = jnp.ones((LANES,), jnp.float32)

    plsc.subcore_barrier()
    ebase = c * EPAD + s * EPW

    def issue_idx(j, q):
        pltpu.async_copy(eidx_hbm.at[pl.ds(ebase + j * EB, EB)], si[q], sisem.at[q])

    def wait_idx(q):
        pltpu.make_async_copy(eidx_hbm.at[pl.ds(ebase, EB)], si[q], sisem.at[q]).wait()

    def step(j, u, do_prefetch):
        wait_idx(u % IDEPTH)
        pltpu.sync_copy(onesv, acc.at[si[u % IDEPTH]], add=True)
        if do_prefetch:
            issue_idx(j + IDEPTH, u % IDEPTH)

    for q in range(IDEPTH):
        issue_idx(q, q)

    @pl.loop(0, NBLK - IDEPTH, step=IDEPTH)
    def _(j):
        for u in range(IDEPTH):
            step(j + u, u, True)

    for u in range(IDEPTH):
        step(NBLK - IDEPTH + u, u, False)

    plsc.subcore_barrier()
    _dump_acc(acc, out_hbm, c, s)
    plsc.subcore_barrier()


def _spmm_phase(c, s, acc, src_hbm, gidx_hbm, sidx_hbm, out_hbm,
                gi, si, gb, zv, gisem, sisem, gsem):
    _zero_acc(acc, s, zv)
    plsc.subcore_barrier()
    ebase = s * EPW

    def issue_idx(j, q):
        e0 = j * EB
        pltpu.async_copy(gidx_hbm.at[pl.ds(ebase + e0, EB)], gi[q], gisem.at[q])
        pltpu.async_copy(sidx_hbm.at[pl.ds(ebase + e0, EB)], si[q], sisem.at[q])

    def wait_idx(q):
        pltpu.make_async_copy(gidx_hbm.at[pl.ds(ebase, EB)], gi[q], gisem.at[q]).wait()
        pltpu.make_async_copy(sidx_hbm.at[pl.ds(ebase, EB)], si[q], sisem.at[q]).wait()

    def issue_gather(q, p):
        pltpu.async_copy(src_hbm.at[gi[q]], gb[p], gsem.at[p])

    def wait_gather(q, p):
        pltpu.make_async_copy(src_hbm.at[gi[q]], gb[p], gsem.at[p]).wait()

    for q in range(IDEPTH):
        issue_idx(q, q)
    for j in range(LOOK):
        wait_idx(j)
        _xform_idx(gi[j], c)
        issue_gather(j, j % GDEPTH)

    def step(j, u, prefetch, advance):
        if advance:
            wait_idx((u + LOOK) % IDEPTH)
            _xform_idx(gi[(u + LOOK) % IDEPTH], c)
            issue_gather((u + LOOK) % IDEPTH, (u + LOOK) % GDEPTH)
        wait_gather(u % IDEPTH, u % GDEPTH)
        pltpu.sync_copy(gb[u % GDEPTH], acc.at[si[u % IDEPTH]], add=True)
        if prefetch:
            issue_idx(j + IDEPTH, u % IDEPTH)

    @pl.loop(0, NBLK - IDEPTH, step=IDEPTH)
    def _(j):
        for u in range(IDEPTH):
            step(j + u, u, True, True)

    for u in range(IDEPTH):
        j = NBLK - IDEPTH + u
        step(j, j % IDEPTH, False, j + LOOK < NBLK)

    plsc.subcore_barrier()
    _dump_acc(acc, out_hbm, c, s)
    plsc.subcore_barrier()


@functools.partial(jax.jit, static_argnums=(0,))
def _layer(with_deg, usrc, isrc, rows_s, cols_s, rows_g, cols_g, eidx):
    outs = [
        jax.ShapeDtypeStruct((NPAD, 2, HALF), jnp.float32),
        jax.ShapeDtypeStruct((NPAD, 2, HALF), jnp.float32),
    ]
    if with_deg:
        outs.append(jax.ShapeDtypeStruct((NPAD, 2, HALF), jnp.float32))

    @functools.partial(
        pl.kernel,
        out_type=tuple(outs),
        mesh=_MESH,
        compiler_params=_SC_PARAMS,
        scratch_types=[
            pltpu.VMEM_SHARED((NPAD, HALF), jnp.float32),
            [pltpu.VMEM((EB,), jnp.int32)] * IDEPTH,
            [pltpu.VMEM((EB,), jnp.int32)] * IDEPTH,
            [pltpu.VMEM((EB, HALF), jnp.float32)] * GDEPTH,
            pltpu.VMEM((EB, HALF), jnp.float32),
            pltpu.VMEM((ZR, HALF), jnp.float32),
            pltpu.SemaphoreType.DMA((IDEPTH,)),
            pltpu.SemaphoreType.DMA((IDEPTH,)),
            pltpu.SemaphoreType.DMA((GDEPTH,)),
        ],
    )
    def k(usrc_h, isrc_h, rows_s_h, cols_s_h, rows_g_h, cols_g_h, eidx_h,
          *refs):
        out_refs, (acc, gi, si, gb, onesv, zv, gisem, sisem, gsem) = (
            refs[:-9], refs[-9:])
        c = lax.axis_index("c")
        s = lax.axis_index("s")
        _zero_fill(zv, ZR, HALF)
        if with_deg:
            us_out, is_out, cnt_out = out_refs
            _deg_phase(c, s, acc, eidx_h, cnt_out, si, onesv, zv, sisem)
        else:
            us_out, is_out = out_refs
        _spmm_phase(c, s, acc, isrc_h, cols_g_h, rows_s_h, us_out,
                    gi, si, gb, zv, gisem, sisem, gsem)
        _spmm_phase(c, s, acc, usrc_h, rows_g_h, cols_s_h, is_out,
                    gi, si, gb, zv, gisem, sisem, gsem)

    return k(usrc, isrc, rows_s, cols_s, rows_g, cols_g, eidx)


NH = N // 2
NHP = NPAD // 2
BR2 = 3128
NB2 = 8


@jax.jit
def _init_slice0(x):
    def body(in_ref, out_ref):
        out_ref[0] = in_ref[...]

    return pl.pallas_call(
        body,
        grid=(NB2,),
        in_specs=[pl.BlockSpec((BR2, 128), lambda i: (i, 0))],
        out_specs=pl.BlockSpec((1, BR2, 128), lambda i: (0, i, 0)),
        out_shape=jax.ShapeDtypeStruct((3, NH, 128), jnp.float32),
    )(x)


@functools.partial(jax.jit, static_argnums=(0, 1))
def _scale(layer, co, sb, seg, cnt, base):
    def body(sb_ref, seg_ref, cnt_ref, base_ref, out_ref, gs_ref):
        del sb_ref
        lane = lax.broadcasted_iota(jnp.int32, (BR2, 128), 1)
        d_even = cnt_ref[:, co:co + 1]
        d_odd = cnt_ref[:, 64 + co:64 + co + 1]
        d = jnp.where(lane < 64, d_even, d_odd)
        d = jnp.where(d == 0.0, 1.0, d)
        r = seg_ref[...] / d + ALPHA * base_ref[...]
        out_ref[0] = r
        gs_ref[...] = r

    return pl.pallas_call(
        body,
        grid=(NB2,),
        in_specs=[
            pl.BlockSpec(memory_space=pl.ANY),
            pl.BlockSpec((BR2, 128), lambda i: (i, 0)),
            pl.BlockSpec((BR2, 128), lambda i: (i, 0)),
            pl.BlockSpec((BR2, 128), lambda i: (i, 0)),
        ],
        out_specs=[
            pl.BlockSpec((1, BR2, 128), lambda i: (layer, i, 0)),
            pl.BlockSpec((BR2, 128), lambda i: (i, 0)),
        ],
        out_shape=[
            jax.ShapeDtypeStruct((3, NH, 128), jnp.float32),
            jax.ShapeDtypeStruct((NH, 128), jnp.float32),
        ],
        input_output_aliases={0: 0},
    )(sb, seg, cnt, base)


def kernel(init_user_embeds, init_item_embeds, rows, cols):
    rows32 = rows.astype(jnp.int32)
    cols32 = cols.astype(jnp.int32)
    padN = jnp.full((EPAD - E,), N, jnp.int32)
    pad0 = jnp.zeros((EPAD - E,), jnp.int32)
    rows_s = jnp.concatenate([rows32, padN])
    cols_s = jnp.concatenate([cols32, padN])
    rows_g = jnp.concatenate([rows32, pad0])
    cols_g = jnp.concatenate([cols32, pad0])
    eidx = jnp.concatenate([rows_s, cols_s])

    u0v = init_user_embeds.reshape(2 * N, HALF)
    i0v = init_item_embeds.reshape(2 * N, HALF)
    u0r = init_user_embeds.reshape(NH, 128)
    i0r = init_item_embeds.reshape(NH, 128)
    sbu = _init_slice0(u0r)
    sbi = _init_slice0(i0r)

    us, is_, cnt = _layer(True, u0v, i0v, rows_s, cols_s, rows_g, cols_g,
                          eidx)
    cntr = cnt.reshape(NHP, 128)
    gsu, gsi = u0v, i0v
    for layer in (1, 2):
        if layer == 2:
            us, is_ = _layer(False, gsu, gsi, rows_s, cols_s, rows_g, cols_g,
                             eidx)
        sbu, gu = _scale(layer, 0, sbu, us.reshape(NHP, 128), cntr, u0r)
        sbi, gi = _scale(layer, 32, sbi, is_.reshape(NHP, 128), cntr, i0r)
        gsu = gu.reshape(2 * N, HALF)
        gsi = gi.reshape(2 * N, HALF)
    return sbu.reshape(3, N, D), sbi.reshape(3, N, D)


# speedup vs baseline: 19.3793x; 1.0098x over previous
import functools

import jax
import jax.numpy as jnp
from jax import lax
from jax.experimental import pallas as pl
from jax.experimental.pallas import tpu as pltpu
from jax.experimental.pallas import tpu_sc as plsc

N = 50000
NPAD = 50048
E = 800000
D = 64
HALF = 32
NSUB = 16
ALPHA = 0.1
LANES = 16

EB = 128
NBLK = 392
EPW = NBLK * EB
EPAD = NSUB * EPW
IDEPTH = 8
GDEPTH = 4
LOOK = 3
RPS = NPAD // NSUB
ZR = 136

_MESH = plsc.VectorSubcoreMesh(core_axis_name="c", subcore_axis_name="s")
_SC_PARAMS = pltpu.CompilerParams(use_tc_tiling_on_sc=False)


def _zero_fill(buf, rows, width):
    @pl.loop(0, rows)
    def _(r):
        for k in range(width // LANES):
            buf[r, pl.ds(k * LANES, LANES)] = jnp.zeros((LANES,), jnp.float32)


def _zero_acc(acc, s, zv):
    @pl.loop(0, RPS // ZR)
    def _(j):
        pltpu.sync_copy(zv, acc.at[pl.ds(s * RPS + j * ZR, ZR)])


def _xform_idx(gi_q, c):
    for k in range(EB // LANES):
        v = jnp.minimum(gi_q[pl.ds(k * LANES, LANES)], N - 1)
        gi_q[pl.ds(k * LANES, LANES)] = v + v + c


def _dump_acc(acc, out_hbm, c, s):
    pltpu.sync_copy(
        acc.at[pl.ds(s * RPS, RPS)],
        out_hbm.at[pl.ds(s * RPS, RPS), c],
    )


def _deg_phase(c, s, acc, eidx_hbm, out_hbm, si, onesv, zv, sisem):
    _zero_acc(acc, s, zv)

    @pl.loop(0, EB)
    def _(r):
        for k in range(HALF // LANES):
            onesv[r, pl.ds(k * LANES, LANES)] = jnp.ones((LANES,), jnp.float32)

    plsc.subcore_barrier()
    ebase = c * EPAD + s * EPW

    def run(idx_hbm):
        def issue_idx(j, q):
            pltpu.async_copy(idx_hbm.at[pl.ds(ebase + j * EB, EB)], si[q],
                             sisem.at[q])

        def wait_idx(q):
            pltpu.make_async_copy(idx_hbm.at[pl.ds(ebase, EB)], si[q],
                                  sisem.at[q]).wait()

        def step(j, u, do_prefetch):
            wait_idx(u % IDEPTH)
            pltpu.sync_copy(onesv, acc.at[si[u % IDEPTH]], add=True)
            if do_prefetch:
                issue_idx(j + IDEPTH, u % IDEPTH)

        for q in range(IDEPTH):
            issue_idx(q, q)

        @pl.loop(0, NBLK - IDEPTH, step=IDEPTH)
        def _(j):
            for u in range(IDEPTH):
                step(j + u, u, True)

        for u in range(IDEPTH):
            step(NBLK - IDEPTH + u, u, False)

    run(eidx_hbm)

    plsc.subcore_barrier()
    _dump_acc(acc, out_hbm, c, s)
    plsc.subcore_barrier()


def _spmm_phase(c, s, acc, src_hbm, eidx_hbm, goff, soff, out_hbm,
                gi, si, gb, zv, gisem, sisem, gsem):
    _zero_acc(acc, s, zv)
    plsc.subcore_barrier()
    gbase = goff + s * EPW
    sbase = soff + s * EPW

    def issue_idx(j, q):
        e0 = j * EB
        pltpu.async_copy(eidx_hbm.at[pl.ds(gbase + e0, EB)], gi[q], gisem.at[q])
        pltpu.async_copy(eidx_hbm.at[pl.ds(sbase + e0, EB)], si[q], sisem.at[q])

    def wait_idx(q):
        pltpu.make_async_copy(eidx_hbm.at[pl.ds(gbase, EB)], gi[q], gisem.at[q]).wait()
        pltpu.make_async_copy(eidx_hbm.at[pl.ds(sbase, EB)], si[q], sisem.at[q]).wait()

    def issue_gather(q, p):
        pltpu.async_copy(src_hbm.at[gi[q]], gb[p], gsem.at[p])

    def wait_gather(q, p):
        pltpu.make_async_copy(src_hbm.at[gi[q]], gb[p], gsem.at[p]).wait()

    for q in range(IDEPTH):
        issue_idx(q, q)
    for j in range(LOOK):
        wait_idx(j)
        _xform_idx(gi[j], c)
        issue_gather(j, j % GDEPTH)

    def step(j, u, prefetch, advance):
        if advance:
            wait_idx((u + LOOK) % IDEPTH)
            _xform_idx(gi[(u + LOOK) % IDEPTH], c)
            issue_gather((u + LOOK) % IDEPTH, (u + LOOK) % GDEPTH)
        wait_gather(u % IDEPTH, u % GDEPTH)
        pltpu.sync_copy(gb[u % GDEPTH], acc.at[si[u % IDEPTH]], add=True)
        if prefetch:
            issue_idx(j + IDEPTH, u % IDEPTH)

    @pl.loop(0, NBLK - IDEPTH, step=IDEPTH)
    def _(j):
        for u in range(IDEPTH):
            step(j + u, u, True, True)

    for u in range(IDEPTH):
        j = NBLK - IDEPTH + u
        step(j, j % IDEPTH, False, j + LOOK < NBLK)

    plsc.subcore_barrier()
    _dump_acc(acc, out_hbm, c, s)
    plsc.subcore_barrier()


@functools.partial(jax.jit, static_argnums=(0,))
def _layer(with_deg, usrc, isrc, eidx):
    outs = [
        jax.ShapeDtypeStruct((NPAD, 2, HALF), jnp.float32),
        jax.ShapeDtypeStruct((NPAD, 2, HALF), jnp.float32),
    ]
    if with_deg:
        outs.append(jax.ShapeDtypeStruct((NPAD, 2, HALF), jnp.float32))

    @functools.partial(
        pl.kernel,
        out_type=tuple(outs),
        mesh=_MESH,
        compiler_params=_SC_PARAMS,
        scratch_types=[
            pltpu.VMEM_SHARED((NPAD, HALF), jnp.float32),
            [pltpu.VMEM((EB,), jnp.int32)] * IDEPTH,
            [pltpu.VMEM((EB,), jnp.int32)] * IDEPTH,
            [pltpu.VMEM((EB, HALF), jnp.float32)] * GDEPTH,
            pltpu.VMEM((EB, HALF), jnp.float32),
            pltpu.VMEM((ZR, HALF), jnp.float32),
            pltpu.SemaphoreType.DMA((IDEPTH,)),
            pltpu.SemaphoreType.DMA((IDEPTH,)),
            pltpu.SemaphoreType.DMA((GDEPTH,)),
        ],
    )
    def k(usrc_h, isrc_h, eidx_h, *refs):
        out_refs, (acc, gi, si, gb, onesv, zv, gisem, sisem, gsem) = (
            refs[:-9], refs[-9:])
        c = lax.axis_index("c")
        s = lax.axis_index("s")
        _zero_fill(zv, ZR, HALF)
        if with_deg:
            us_out, is_out, cnt_out = out_refs
            _deg_phase(c, s, acc, eidx_h, cnt_out, si, onesv, zv, sisem)
        else:
            us_out, is_out = out_refs
        _spmm_phase(c, s, acc, isrc_h, eidx_h, EPAD, 0, us_out,
                    gi, si, gb, zv, gisem, sisem, gsem)
        _spmm_phase(c, s, acc, usrc_h, eidx_h, 0, EPAD, is_out,
                    gi, si, gb, zv, gisem, sisem, gsem)

    return k(usrc, isrc, eidx)


NH = N // 2
NHP = NPAD // 2
BR2 = 3128
NB2 = 8


@jax.jit
def _init_slice0(x):
    def body(in_ref, out_ref):
        out_ref[0] = in_ref[...]

    return pl.pallas_call(
        body,
        grid=(NB2,),
        in_specs=[pl.BlockSpec((BR2, 128), lambda i: (i, 0))],
        out_specs=pl.BlockSpec((1, BR2, 128), lambda i: (0, i, 0)),
        out_shape=jax.ShapeDtypeStruct((3, NH, 128), jnp.float32),
    )(x)


@functools.partial(jax.jit, static_argnums=(0, 1))
def _scale(layer, co, sb, seg, cnt, base):
    def body(sb_ref, seg_ref, cnt_ref, base_ref, out_ref, gs_ref):
        del sb_ref
        lane = lax.broadcasted_iota(jnp.int32, (BR2, 128), 1)
        d_even = cnt_ref[:, co:co + 1]
        d_odd = cnt_ref[:, 64 + co:64 + co + 1]
        d = jnp.where(lane < 64, d_even, d_odd)
        d = jnp.where(d == 0.0, 1.0, d)
        r = seg_ref[...] / d + ALPHA * base_ref[...]
        out_ref[0] = r
        gs_ref[...] = r

    return pl.pallas_call(
        body,
        grid=(NB2,),
        in_specs=[
            pl.BlockSpec(memory_space=pl.ANY),
            pl.BlockSpec((BR2, 128), lambda i: (i, 0)),
            pl.BlockSpec((BR2, 128), lambda i: (i, 0)),
            pl.BlockSpec((BR2, 128), lambda i: (i, 0)),
        ],
        out_specs=[
            pl.BlockSpec((1, BR2, 128), lambda i: (layer, i, 0)),
            pl.BlockSpec((BR2, 128), lambda i: (i, 0)),
        ],
        out_shape=[
            jax.ShapeDtypeStruct((3, NH, 128), jnp.float32),
            jax.ShapeDtypeStruct((NH, 128), jnp.float32),
        ],
        input_output_aliases={0: 0},
    )(sb, seg, cnt, base)


def kernel(init_user_embeds, init_item_embeds, rows, cols):
    rows32 = rows.astype(jnp.int32)
    cols32 = cols.astype(jnp.int32)
    padN = jnp.full((EPAD - E,), N, jnp.int32)
    eidx = jnp.concatenate([rows32, padN, cols32, padN])

    u0v = init_user_embeds.reshape(2 * N, HALF)
    i0v = init_item_embeds.reshape(2 * N, HALF)
    u0r = init_user_embeds.reshape(NH, 128)
    i0r = init_item_embeds.reshape(NH, 128)
    sbu = _init_slice0(u0r)
    sbi = _init_slice0(i0r)

    us, is_, cnt = _layer(True, u0v, i0v, eidx)
    cntr = cnt.reshape(NHP, 128)
    gsu, gsi = u0v, i0v
    for layer in (1, 2):
        if layer == 2:
            us, is_ = _layer(False, gsu, gsi, eidx)
        sbu, gu = _scale(layer, 0, sbu, us.reshape(NHP, 128), cntr, u0r)
        sbi, gi = _scale(layer, 32, sbi, is_.reshape(NHP, 128), cntr, i0r)
        gsu = gu.reshape(2 * N, HALF)
        gsi = gi.reshape(2 * N, HALF)
    return sbu.reshape(3, N, D), sbi.reshape(3, N, D)


# speedup vs baseline: 19.6250x; 1.0127x over previous
import functools

import jax
import jax.numpy as jnp
from jax import lax
from jax.experimental import pallas as pl
from jax.experimental.pallas import tpu as pltpu
from jax.experimental.pallas import tpu_sc as plsc

N = 50000
NPAD = 50048
E = 800000
D = 64
HALF = 32
NSUB = 16
ALPHA = 0.1
LANES = 16

EB = 128
NBLK = 392
EPW = NBLK * EB
EPAD = NSUB * EPW
IDEPTH = 8
GDEPTH = 4
LOOK = 3
SD = 4
WS = 3
PERIOD = 8
RPS = NPAD // NSUB
ZR = 136

_MESH = plsc.VectorSubcoreMesh(core_axis_name="c", subcore_axis_name="s")
_SC_PARAMS = pltpu.CompilerParams(use_tc_tiling_on_sc=False)


def _zero_fill(buf, rows, width):
    @pl.loop(0, rows)
    def _(r):
        for k in range(width // LANES):
            buf[r, pl.ds(k * LANES, LANES)] = jnp.zeros((LANES,), jnp.float32)


def _zero_acc(acc, s, zv):
    @pl.loop(0, RPS // ZR)
    def _(j):
        pltpu.sync_copy(zv, acc.at[pl.ds(s * RPS + j * ZR, ZR)])


def _xform_idx(gi_q, c):
    for k in range(EB // LANES):
        v = jnp.minimum(gi_q[pl.ds(k * LANES, LANES)], N - 1)
        gi_q[pl.ds(k * LANES, LANES)] = v + v + c


def _dump_acc(acc, out_hbm, c, s):
    pltpu.sync_copy(
        acc.at[pl.ds(s * RPS, RPS)],
        out_hbm.at[pl.ds(s * RPS, RPS), c],
    )


def _deg_phase(c, s, acc, eidx_hbm, out_hbm, si, onesv, zv, sisem, scsem):
    _zero_acc(acc, s, zv)

    @pl.loop(0, EB)
    def _(r):
        for k in range(HALF // LANES):
            onesv[r, pl.ds(k * LANES, LANES)] = jnp.ones((LANES,), jnp.float32)

    plsc.subcore_barrier()
    ebase = c * EPAD + s * EPW

    def issue_idx(j, q):
        pltpu.async_copy(eidx_hbm.at[pl.ds(ebase + j * EB, EB)], si[q],
                         sisem.at[q])

    def wait_idx(q):
        pltpu.make_async_copy(eidx_hbm.at[pl.ds(ebase, EB)], si[q],
                              sisem.at[q]).wait()

    def issue_scatter(r):
        pltpu.async_copy(onesv, acc.at[si[r % IDEPTH]], scsem.at[r % SD],
                         add=True)

    def wait_scatter(r):
        pltpu.make_async_copy(onesv, acc.at[si[r % IDEPTH]],
                              scsem.at[r % SD]).wait()

    for q in range(IDEPTH):
        issue_idx(q, q)
    for j in range(WS):
        wait_idx(j % IDEPTH)
        issue_scatter(j)

    @pl.loop(WS, NBLK - IDEPTH + WS, step=IDEPTH)
    def _(jv):
        for u in range(IDEPTH):
            r = WS + u
            wait_scatter(r - WS)
            issue_idx(jv + u + IDEPTH - WS, (r + IDEPTH - WS) % IDEPTH)
            wait_idx(r % IDEPTH)
            issue_scatter(r)

    for j in range(NBLK - IDEPTH + WS, NBLK):
        wait_scatter(j - WS)
        wait_idx(j % IDEPTH)
        issue_scatter(j)
    for j in range(NBLK - WS, NBLK):
        wait_scatter(j)

    plsc.subcore_barrier()
    _dump_acc(acc, out_hbm, c, s)
    plsc.subcore_barrier()


def _spmm_phase(c, s, acc, src_hbm, eidx_hbm, goff, soff, out_hbm,
                gi, si, gb, zv, gisem, sisem, gsem, scsem):
    _zero_acc(acc, s, zv)
    plsc.subcore_barrier()
    gbase = goff + s * EPW
    sbase = soff + s * EPW

    def issue_idx(j, q):
        e0 = j * EB
        pltpu.async_copy(eidx_hbm.at[pl.ds(gbase + e0, EB)], gi[q], gisem.at[q])
        pltpu.async_copy(eidx_hbm.at[pl.ds(sbase + e0, EB)], si[q], sisem.at[q])

    def wait_idx(q):
        pltpu.make_async_copy(eidx_hbm.at[pl.ds(gbase, EB)], gi[q], gisem.at[q]).wait()
        pltpu.make_async_copy(eidx_hbm.at[pl.ds(sbase, EB)], si[q], sisem.at[q]).wait()

    def issue_gather(r):
        pltpu.async_copy(src_hbm.at[gi[r % IDEPTH]], gb[r % GDEPTH],
                         gsem.at[r % GDEPTH])

    def wait_gather(r):
        pltpu.make_async_copy(src_hbm.at[gi[r % IDEPTH]], gb[r % GDEPTH],
                              gsem.at[r % GDEPTH]).wait()

    def issue_scatter(r):
        pltpu.async_copy(gb[r % GDEPTH], acc.at[si[r % IDEPTH]],
                         scsem.at[r % SD], add=True)

    def wait_scatter(r):
        pltpu.make_async_copy(gb[r % GDEPTH], acc.at[si[r % IDEPTH]],
                              scsem.at[r % SD]).wait()

    for q in range(IDEPTH):
        issue_idx(q, q)
    for t in range(LOOK):
        wait_idx(t)
        _xform_idx(gi[t], c)
        issue_gather(t)
    for j in range(WS):
        wait_idx(j + LOOK)
        _xform_idx(gi[j + LOOK], c)
        issue_gather(j + LOOK)
        wait_gather(j)
        issue_scatter(j)

    @pl.loop(WS, NBLK - IDEPTH + WS, step=PERIOD)
    def _(jv):
        for u in range(PERIOD):
            j = jv + u
            r = WS + u
            wait_scatter(r - WS)
            issue_idx(j + IDEPTH - WS, (r + IDEPTH - WS) % IDEPTH)
            wait_idx((r + LOOK) % IDEPTH)
            _xform_idx(gi[(r + LOOK) % IDEPTH], c)
            issue_gather(r + LOOK)
            wait_gather(r)
            issue_scatter(r)

    for j in range(NBLK - IDEPTH + WS, NBLK):
        wait_scatter(j - WS)
        if j + LOOK < NBLK:
            wait_idx((j + LOOK) % IDEPTH)
            _xform_idx(gi[(j + LOOK) % IDEPTH], c)
            issue_gather(j + LOOK)
        wait_gather(j)
        issue_scatter(j)
    for j in range(NBLK - WS, NBLK):
        wait_scatter(j)

    plsc.subcore_barrier()
    _dump_acc(acc, out_hbm, c, s)
    plsc.subcore_barrier()


@functools.partial(jax.jit, static_argnums=(0,))
def _layer(with_deg, usrc, isrc, eidx):
    outs = [
        jax.ShapeDtypeStruct((NPAD, 2, HALF), jnp.float32),
        jax.ShapeDtypeStruct((NPAD, 2, HALF), jnp.float32),
    ]
    if with_deg:
        outs.append(jax.ShapeDtypeStruct((NPAD, 2, HALF), jnp.float32))

    @functools.partial(
        pl.kernel,
        out_type=tuple(outs),
        mesh=_MESH,
        compiler_params=_SC_PARAMS,
        scratch_types=[
            pltpu.VMEM_SHARED((NPAD, HALF), jnp.float32),
            [pltpu.VMEM((EB,), jnp.int32)] * IDEPTH,
            [pltpu.VMEM((EB,), jnp.int32)] * IDEPTH,
            [pltpu.VMEM((EB, HALF), jnp.float32)] * GDEPTH,
            pltpu.VMEM((EB, HALF), jnp.float32),
            pltpu.VMEM((ZR, HALF), jnp.float32),
            pltpu.SemaphoreType.DMA((IDEPTH,)),
            pltpu.SemaphoreType.DMA((IDEPTH,)),
            pltpu.SemaphoreType.DMA((GDEPTH,)),
            pltpu.SemaphoreType.DMA((SD,)),
        ],
    )
    def k(usrc_h, isrc_h, eidx_h, *refs):
        out_refs, (acc, gi, si, gb, onesv, zv, gisem, sisem, gsem, scsem) = (
            refs[:-10], refs[-10:])
        c = lax.axis_index("c")
        s = lax.axis_index("s")
        _zero_fill(zv, ZR, HALF)
        if with_deg:
            us_out, is_out, cnt_out = out_refs
            _deg_phase(c, s, acc, eidx_h, cnt_out, si, onesv, zv, sisem,
                       scsem)
        else:
            us_out, is_out = out_refs
        _spmm_phase(c, s, acc, isrc_h, eidx_h, EPAD, 0, us_out,
                    gi, si, gb, zv, gisem, sisem, gsem, scsem)
        _spmm_phase(c, s, acc, usrc_h, eidx_h, 0, EPAD, is_out,
                    gi, si, gb, zv, gisem, sisem, gsem, scsem)

    return k(usrc, isrc, eidx)


NH = N // 2
NHP = NPAD // 2
BR2 = 3128
NB2 = 8


@jax.jit
def _init_slice0(x):
    def body(in_ref, out_ref):
        out_ref[0] = in_ref[...]

    return pl.pallas_call(
        body,
        grid=(NB2,),
        in_specs=[pl.BlockSpec((BR2, 128), lambda i: (i, 0))],
        out_specs=pl.BlockSpec((1, BR2, 128), lambda i: (0, i, 0)),
        out_shape=jax.ShapeDtypeStruct((3, NH, 128), jnp.float32),
    )(x)


@functools.partial(jax.jit, static_argnums=(0, 1))
def _scale(layer, co, sb, seg, cnt, base):
    def body(sb_ref, seg_ref, cnt_ref, base_ref, out_ref, gs_ref):
        del sb_ref
        lane = lax.broadcasted_iota(jnp.int32, (BR2, 128), 1)
        d_even = cnt_ref[:, co:co + 1]
        d_odd = cnt_ref[:, 64 + co:64 + co + 1]
        d = jnp.where(lane < 64, d_even, d_odd)
        d = jnp.where(d == 0.0, 1.0, d)
        r = seg_ref[...] / d + ALPHA * base_ref[...]
        out_ref[0] = r
        gs_ref[...] = r

    return pl.pallas_call(
        body,
        grid=(NB2,),
        in_specs=[
            pl.BlockSpec(memory_space=pl.ANY),
            pl.BlockSpec((BR2, 128), lambda i: (i, 0)),
            pl.BlockSpec((BR2, 128), lambda i: (i, 0)),
            pl.BlockSpec((BR2, 128), lambda i: (i, 0)),
        ],
        out_specs=[
            pl.BlockSpec((1, BR2, 128), lambda i: (layer, i, 0)),
            pl.BlockSpec((BR2, 128), lambda i: (i, 0)),
        ],
        out_shape=[
            jax.ShapeDtypeStruct((3, NH, 128), jnp.float32),
            jax.ShapeDtypeStruct((NH, 128), jnp.float32),
        ],
        input_output_aliases={0: 0},
    )(sb, seg, cnt, base)


def kernel(init_user_embeds, init_item_embeds, rows, cols):
    rows32 = rows.astype(jnp.int32)
    cols32 = cols.astype(jnp.int32)
    padN = jnp.full((EPAD - E,), N, jnp.int32)
    eidx = jnp.concatenate([rows32, padN, cols32, padN])

    u0v = init_user_embeds.reshape(2 * N, HALF)
    i0v = init_item_embeds.reshape(2 * N, HALF)
    u0r = init_user_embeds.reshape(NH, 128)
    i0r = init_item_embeds.reshape(NH, 128)
    sbu = _init_slice0(u0r)
    sbi = _init_slice0(i0r)

    us, is_, cnt = _layer(True, u0v, i0v, eidx)
    cntr = cnt.reshape(NHP, 128)
    gsu, gsi = u0v, i0v
    for layer in (1, 2):
        if layer == 2:
            us, is_ = _layer(False, gsu, gsi, eidx)
        sbu, gu = _scale(layer, 0, sbu, us.reshape(NHP, 128), cntr, u0r)
        sbi, gi = _scale(layer, 32, sbi, is_.reshape(NHP, 128), cntr, i0r)
        gsu = gu.reshape(2 * N, HALF)
        gsi = gi.reshape(2 * N, HALF)
    return sbu.reshape(3, N, D), sbi.reshape(3, N, D)
